# Initial kernel scaffold; baseline (speedup 1.0000x reference)
import jax
import jax.numpy as jnp
from jax.experimental import pallas as pl


def kernel(x_cat, x_cont, tables, W1, b1, W2, b2, W3, b3):
    raise NotImplementedError("write your pallas kernel here")


# speedup vs baseline: 7.4520x; 7.4520x over previous
import functools

import jax
import jax.numpy as jnp
from jax.experimental import pallas as pl
from jax.experimental.pallas import tpu as pltpu
from jax.experimental.pallas import tpu_sc as plsc

N_FIELDS = 26
VOCAB = 100000
EMB = 16
N_CONT = 13
BATCH = 16384
OUT = 2
EMB_TOTAL = N_FIELDS * EMB
TOTAL_IDX = BATCH * N_FIELDS
WINDOW = 128
NUM_WINDOWS = TOTAL_IDX // WINDOW


@jax.jit
def _sc_gather(tables_flat, gidx2d):
    mesh = plsc.VectorSubcoreMesh(core_axis_name="core", subcore_axis_name="subcore")

    @functools.partial(
        pl.kernel,
        out_type=jax.ShapeDtypeStruct((TOTAL_IDX, EMB), jnp.float32),
        mesh=mesh,
        compiler_params=pltpu.CompilerParams(use_tc_tiling_on_sc=False),
    )
    def k(tab_hbm, idx_hbm, out_hbm):
        def body(i_vmem, o_vmem):
            pltpu.sync_copy(tab_hbm.at[i_vmem.at[0]], o_vmem)

        pltpu.emit_pipeline(
            body,
            grid=(NUM_WINDOWS,),
            in_specs=[pl.BlockSpec((1, WINDOW), lambda i: (0, i))],
            out_specs=[pl.BlockSpec((WINDOW, EMB), lambda i: (i, 0))],
            core_axis_name=("core", "subcore"),
            dimension_semantics=(pltpu.PARALLEL,),
        )(idx_hbm, out_hbm)

    return k(tables_flat, gidx2d)


def _mlp_body(e_ref, c_ref, w1e_ref, w1c_ref, b1_ref, w2_ref, b2_ref,
              w3_ref, b3_ref, o_ref):
    h = jnp.dot(e_ref[...], w1e_ref[...], preferred_element_type=jnp.float32)
    h = h + jnp.dot(c_ref[...], w1c_ref[...], preferred_element_type=jnp.float32)
    h = jnp.maximum(h + b1_ref[...], 0.0)
    h = jnp.dot(h, w2_ref[...], preferred_element_type=jnp.float32) + b2_ref[...]
    h = jnp.maximum(h, 0.0)
    o_ref[...] = jnp.dot(h, w3_ref[...], preferred_element_type=jnp.float32) + b3_ref[...]


def _tc_mlp(embs, x_cont, w1e, w1c, b1, w2, b2, w3, b3):
    blk = 2048
    grid = BATCH // blk
    return pl.pallas_call(
        _mlp_body,
        grid=(grid,),
        in_specs=[
            pl.BlockSpec((blk, EMB_TOTAL), lambda i: (i, 0)),
            pl.BlockSpec((blk, N_CONT), lambda i: (i, 0)),
            pl.BlockSpec((EMB_TOTAL, 128), lambda i: (0, 0)),
            pl.BlockSpec((N_CONT, 128), lambda i: (0, 0)),
            pl.BlockSpec((1, 128), lambda i: (0, 0)),
            pl.BlockSpec((128, 64), lambda i: (0, 0)),
            pl.BlockSpec((1, 64), lambda i: (0, 0)),
            pl.BlockSpec((64, OUT), lambda i: (0, 0)),
            pl.BlockSpec((1, OUT), lambda i: (0, 0)),
        ],
        out_specs=pl.BlockSpec((blk, OUT), lambda i: (i, 0)),
        out_shape=jax.ShapeDtypeStruct((BATCH, OUT), jnp.float32),
    )(embs, x_cont, w1e, w1c, b1, w2, b2, w3, b3)


def kernel(x_cat, x_cont, tables, W1, b1, W2, b2, W3, b3):
    offsets = jnp.arange(N_FIELDS, dtype=jnp.int32) * VOCAB
    gidx = (x_cat.astype(jnp.int32) + offsets[None, :]).reshape(1, TOTAL_IDX)
    tables_flat = tables.reshape(N_FIELDS * VOCAB, EMB)
    rows = _sc_gather(tables_flat, gidx)
    embs = rows.reshape(BATCH, EMB_TOTAL)
    return _tc_mlp(
        embs, x_cont,
        W1[:EMB_TOTAL], W1[EMB_TOTAL:],
        b1.reshape(1, 128), W2, b2.reshape(1, 64), W3, b3.reshape(1, OUT),
    )
